# baseline (speedup 1.0000x reference)
import functools

import jax
import jax.numpy as jnp
from jax import lax
from jax.experimental import pallas as pl
from jax.experimental.pallas import tpu as pltpu
from jax.experimental.pallas import tpu_sc as plsc

KEY_DIM = 128
MEM_SIZE = 65536
CHOOSE_K = 256
FC_DIM = 1024
BATCH = 256

MEM_CHUNK = 8192
N_CHUNKS = MEM_SIZE // MEM_CHUNK
GROUP = 16
N_GROUPS = MEM_SIZE // GROUP

NC = 2
NS = 16
NW = NC * NS
RPT = MEM_SIZE // NW
BPT = BATCH // NW
LANES = 16


def _q_kernel(x_ref, w1_ref, b1_ref, w2_ref, b2_ref, q_ref):
    h = jnp.maximum(
        jnp.dot(x_ref[...], w1_ref[...], preferred_element_type=jnp.float32)
        + b1_ref[...], 0.0)
    q = jnp.dot(h, w2_ref[...], preferred_element_type=jnp.float32) + b2_ref[...]
    qn = jnp.sqrt(jnp.sum(q * q, axis=1, keepdims=True))
    q_ref[...] = q / (qn + 1e-8)


def _sim_kernel(q_ref, mk_ref, sim_ref):
    sim_ref[...] = jax.lax.dot_general(
        q_ref[...], mk_ref[...], (((1,), (1,)), ((), ())),
        preferred_element_type=jnp.float32)


def _update_kernel(mk_hbm, mv_hbm, ma_hbm, wk_hbm, wi_hbm, lab_hbm,
                   tv_hbm, ti_hbm,
                   nk_hbm, nv_hbm, na_hbm, post_hbm,
                   vals_v, ages_v, wi_v, lab_v, tv_v, ti_v, mvg_v, p_v,
                   sem_keys, sem_g):
    wid = lax.axis_index("s") * NC + lax.axis_index("c")
    base = wid * RPT

    keys_cp = pltpu.async_copy(
        mk_hbm.at[pl.ds(base, RPT)], nk_hbm.at[pl.ds(base, RPT)], sem_keys)

    pltpu.sync_copy(mv_hbm.at[pl.ds(base, RPT)], vals_v)
    pltpu.sync_copy(ma_hbm.at[pl.ds(base, RPT)], ages_v)
    pltpu.sync_copy(wi_hbm, wi_v)
    pltpu.sync_copy(lab_hbm, lab_v)

    def age_body(i, _):
        ages_v[pl.ds(i * LANES, LANES)] = ages_v[pl.ds(i * LANES, LANES)] + 1.0
        return 0
    lax.fori_loop(0, RPT // LANES, age_body, 0)

    lane_iota = lax.iota(jnp.int32, LANES)

    def scat_body(k, _):
        iv = wi_v[pl.ds(k * LANES, LANES)]
        lv = lab_v[pl.ds(k * LANES, LANES)]
        for lane in range(LANES):
            idx = iv[lane]

            @pl.when((idx >= base) & (idx < base + RPT))
            def _():
                liv = idx - base
                off = liv & ~(LANES - 1)
                m = lane_iota == (liv - off)
                vv = vals_v[pl.ds(off, LANES)]
                vals_v[pl.ds(off, LANES)] = jnp.where(
                    m, jnp.broadcast_to(lv[lane], (LANES,)), vv)
                av = ages_v[pl.ds(off, LANES)]
                ages_v[pl.ds(off, LANES)] = jnp.where(m, 0.0, av)
        return 0
    lax.fori_loop(0, CHOOSE_K // LANES, scat_body, 0)

    pltpu.sync_copy(vals_v, nv_hbm.at[pl.ds(base, RPT)])
    pltpu.sync_copy(ages_v, na_hbm.at[pl.ds(base, RPT)])

    keys_cp.wait()

    def key_body(k, _):
        jv = wi_v[pl.ds(k * LANES, LANES)]
        for lane in range(LANES):
            idx = jv[lane]
            j = k * LANES + lane

            @pl.when((idx >= base) & (idx < base + RPT))
            def _():
                pltpu.sync_copy(wk_hbm.at[j], nk_hbm.at[idx])
        return 0
    lax.fori_loop(0, CHOOSE_K // LANES, key_body, 0)

    posts = jnp.zeros((LANES,), jnp.float32)
    lane_iota = lax.iota(jnp.int32, LANES)
    for r_local in range(BPT):
        r = wid * BPT + r_local
        pltpu.sync_copy(tv_hbm.at[r], tv_v)
        pltpu.sync_copy(ti_hbm.at[r], ti_v)
        g1 = pltpu.async_copy(
            mv_hbm.at[ti_v.at[pl.ds(0, 128)]], mvg_v.at[pl.ds(0, 128)], sem_g)
        g2 = pltpu.async_copy(
            mv_hbm.at[ti_v.at[pl.ds(128, 128)]], mvg_v.at[pl.ds(128, 128)],
            sem_g)
        g1.wait()
        g2.wait()
        mx = tv_v[pl.ds(0, LANES)][0]

        def post_body(k, carry):
            z_acc, a_acc = carry
            tvk = tv_v[pl.ds(k * LANES, LANES)]
            mvk = mvg_v[pl.ds(k * LANES, LANES)].astype(jnp.float32)
            e = jnp.exp(tvk - mx)
            return z_acc + e, a_acc + e * mvk
        z_acc, a_acc = lax.fori_loop(
            0, CHOOSE_K // LANES, post_body,
            (jnp.zeros((LANES,), jnp.float32), jnp.zeros((LANES,), jnp.float32)))
        z_tot = z_acc[0]
        a_tot = a_acc[0]
        for lane in range(1, LANES):
            z_tot = z_tot + z_acc[lane]
            a_tot = a_tot + a_acc[lane]
        pv = (jnp.broadcast_to(a_tot, (LANES,))
              / jnp.broadcast_to(z_tot, (LANES,)))
        posts = jnp.where(lane_iota == r_local, pv, posts)
    p_v[...] = posts
    pltpu.sync_copy(p_v, post_hbm.at[wid])


_update_call = functools.partial(
    pl.kernel,
    out_type=[
        jax.ShapeDtypeStruct((MEM_SIZE, KEY_DIM), jnp.float32),
        jax.ShapeDtypeStruct((MEM_SIZE,), jnp.int32),
        jax.ShapeDtypeStruct((MEM_SIZE,), jnp.float32),
        jax.ShapeDtypeStruct((NW, LANES), jnp.float32),
    ],
    mesh=plsc.VectorSubcoreMesh(core_axis_name="c", subcore_axis_name="s"),
    scratch_types=[
        pltpu.VMEM((RPT,), jnp.int32),
        pltpu.VMEM((RPT,), jnp.float32),
        pltpu.VMEM((CHOOSE_K,), jnp.int32),
        pltpu.VMEM((CHOOSE_K,), jnp.int32),
        pltpu.VMEM((CHOOSE_K,), jnp.float32),
        pltpu.VMEM((CHOOSE_K,), jnp.int32),
        pltpu.VMEM((CHOOSE_K,), jnp.int32),
        pltpu.VMEM((LANES,), jnp.float32),
        pltpu.SemaphoreType.DMA,
        pltpu.SemaphoreType.DMA,
    ],
)(_update_kernel)


def kernel(x, label, W1, b1, W2, b2, mem_keys, mem_values, mem_ages):
    B = x.shape[0]
    xf = x.reshape(B, -1)

    q = pl.pallas_call(
        _q_kernel,
        out_shape=jax.ShapeDtypeStruct((B, KEY_DIM), jnp.float32),
    )(xf, W1, b1.reshape(1, FC_DIM), W2, b2.reshape(1, KEY_DIM))

    sim = pl.pallas_call(
        _sim_kernel,
        grid=(N_CHUNKS,),
        in_specs=[
            pl.BlockSpec((B, KEY_DIM), lambda i: (0, 0)),
            pl.BlockSpec((MEM_CHUNK, KEY_DIM), lambda i: (i, 0)),
        ],
        out_specs=pl.BlockSpec((B, MEM_CHUNK), lambda i: (0, i)),
        out_shape=jax.ShapeDtypeStruct((B, MEM_SIZE), jnp.float32),
    )(q, mem_keys)

    gmax = jnp.max(sim.reshape(B, N_GROUPS, GROUP), axis=-1)
    _, gidx = jax.lax.top_k(gmax, CHOOSE_K)
    cand_idx = (gidx[:, :, None] * GROUP
                + jnp.arange(GROUP, dtype=gidx.dtype)).reshape(B, CHOOSE_K * GROUP)
    cand = jnp.take_along_axis(sim, cand_idx, axis=1)
    topv, topj = jax.lax.top_k(cand, CHOOSE_K)
    topi = jnp.take_along_axis(cand_idx, topj, axis=1)

    nearest = topi[:, 0]
    match = jnp.take(mem_values, nearest, axis=0) == label
    merged = q + jnp.take(mem_keys, nearest, axis=0)
    merged = merged / (jnp.linalg.norm(merged, axis=1, keepdims=True) + 1e-8)
    _, oldest = jax.lax.top_k(mem_ages, B)
    write_idx = jnp.where(match, nearest, oldest).astype(jnp.int32)
    write_key = jnp.where(match[:, None], merged, q)

    jj = jnp.arange(CHOOSE_K, dtype=jnp.int32)
    eq = write_idx[:, None] == write_idx[None, :]
    winner = jnp.max(jnp.where(eq, jj[None, :], -1), axis=1)
    wk_eff = jnp.take(write_key, winner, axis=0)
    lab_eff = jnp.take(label.astype(jnp.int32), winner, axis=0)

    new_keys, new_values, new_ages, post_pad = _update_call(
        mem_keys, mem_values, mem_ages, wk_eff, write_idx, lab_eff,
        topv, topi)
    post_prob = post_pad[:, :BPT].reshape(B)
    return post_prob, new_keys, new_values, new_ages


# speedup vs baseline: 2.2100x; 2.2100x over previous
import functools

import jax
import jax.numpy as jnp
from jax import lax
from jax.experimental import pallas as pl
from jax.experimental.pallas import tpu as pltpu
from jax.experimental.pallas import tpu_sc as plsc

KEY_DIM = 128
MEM_SIZE = 65536
CHOOSE_K = 256
FC_DIM = 1024
BATCH = 256

MEM_CHUNK = 8192
N_CHUNKS = MEM_SIZE // MEM_CHUNK
GROUP = 16
N_GROUPS = MEM_SIZE // GROUP

NC = 2
NS = 16
NW = NC * NS
RPT = MEM_SIZE // NW
BPT = BATCH // NW
LANES = 16
KCH = 256


def _q_kernel(x_ref, w1_ref, b1_ref, w2_ref, b2_ref, q_ref):
    h = jnp.maximum(
        jnp.dot(x_ref[...], w1_ref[...], preferred_element_type=jnp.float32)
        + b1_ref[...], 0.0)
    q = jnp.dot(h, w2_ref[...], preferred_element_type=jnp.float32) + b2_ref[...]
    qn = jnp.sqrt(jnp.sum(q * q, axis=1, keepdims=True))
    q_ref[...] = q / (qn + 1e-8)


def _sim_kernel(q_ref, mk_ref, sim_ref):
    sim_ref[...] = jax.lax.dot_general(
        q_ref[...], mk_ref[...], (((1,), (1,)), ((), ())),
        preferred_element_type=jnp.float32)


def _update_kernel(mk_hbm, mv_hbm, ma_hbm, wk_hbm, wi_hbm, lab_hbm,
                   tv_hbm, ti_hbm,
                   nk_hbm, nv_hbm, na_hbm, post_hbm,
                   vals_v, ages_v, wi_v, lab_v, tv_v, ti_v, mvg_v, p_v,
                   kbuf_v, sem_keys, sem_kout, sem_g):
    wid = lax.axis_index("s") * NC + lax.axis_index("c")
    base = wid * RPT

    n_ch = RPT // KCH
    cin0 = pltpu.async_copy(
        mk_hbm.at[pl.ds(base, KCH)], kbuf_v.at[0], sem_keys)
    cin_prev = cin0
    out_prev = None
    for ci in range(n_ch):
        if out_prev is not None:
            out_prev.wait()
        nxt = None
        if ci + 1 < n_ch:
            nxt = pltpu.async_copy(
                mk_hbm.at[pl.ds(base + (ci + 1) * KCH, KCH)],
                kbuf_v.at[(ci + 1) % 2], sem_keys)
        cin_prev.wait()
        out_prev = pltpu.async_copy(
            kbuf_v.at[ci % 2], nk_hbm.at[pl.ds(base + ci * KCH, KCH)],
            sem_kout)
        cin_prev = nxt

    pltpu.sync_copy(mv_hbm.at[pl.ds(base, RPT)], vals_v)
    pltpu.sync_copy(ma_hbm.at[pl.ds(base, RPT)], ages_v)
    pltpu.sync_copy(wi_hbm, wi_v)
    pltpu.sync_copy(lab_hbm, lab_v)

    def age_body(i, _):
        ages_v[pl.ds(i * LANES, LANES)] = ages_v[pl.ds(i * LANES, LANES)] + 1.0
        return 0
    lax.fori_loop(0, RPT // LANES, age_body, 0)

    lane_iota = lax.iota(jnp.int32, LANES)

    def scat_body(k, _):
        iv = wi_v[pl.ds(k * LANES, LANES)]
        lv = lab_v[pl.ds(k * LANES, LANES)]
        for lane in range(LANES):
            idx = iv[lane]

            @pl.when((idx >= base) & (idx < base + RPT))
            def _():
                liv = idx - base
                off = liv & ~(LANES - 1)
                m = lane_iota == (liv - off)
                vv = vals_v[pl.ds(off, LANES)]
                vals_v[pl.ds(off, LANES)] = jnp.where(
                    m, jnp.broadcast_to(lv[lane], (LANES,)), vv)
                av = ages_v[pl.ds(off, LANES)]
                ages_v[pl.ds(off, LANES)] = jnp.where(m, 0.0, av)
        return 0
    lax.fori_loop(0, CHOOSE_K // LANES, scat_body, 0)

    pltpu.sync_copy(vals_v, nv_hbm.at[pl.ds(base, RPT)])
    pltpu.sync_copy(ages_v, na_hbm.at[pl.ds(base, RPT)])

    out_prev.wait()

    def key_body(k, _):
        jv = wi_v[pl.ds(k * LANES, LANES)]
        for lane in range(LANES):
            idx = jv[lane]
            j = k * LANES + lane

            @pl.when((idx >= base) & (idx < base + RPT))
            def _():
                pltpu.sync_copy(wk_hbm.at[j], nk_hbm.at[idx])
        return 0
    lax.fori_loop(0, CHOOSE_K // LANES, key_body, 0)

    posts = jnp.zeros((LANES,), jnp.float32)
    lane_iota = lax.iota(jnp.int32, LANES)
    for r_local in range(BPT):
        r = wid * BPT + r_local
        pltpu.sync_copy(tv_hbm.at[r], tv_v)
        pltpu.sync_copy(ti_hbm.at[r], ti_v)
        g1 = pltpu.async_copy(
            mv_hbm.at[ti_v.at[pl.ds(0, 128)]], mvg_v.at[pl.ds(0, 128)], sem_g)
        g2 = pltpu.async_copy(
            mv_hbm.at[ti_v.at[pl.ds(128, 128)]], mvg_v.at[pl.ds(128, 128)],
            sem_g)
        g1.wait()
        g2.wait()
        mx = tv_v[pl.ds(0, LANES)][0]

        def post_body(k, carry):
            z_acc, a_acc = carry
            tvk = tv_v[pl.ds(k * LANES, LANES)]
            mvk = mvg_v[pl.ds(k * LANES, LANES)].astype(jnp.float32)
            e = jnp.exp(tvk - mx)
            return z_acc + e, a_acc + e * mvk
        z_acc, a_acc = lax.fori_loop(
            0, CHOOSE_K // LANES, post_body,
            (jnp.zeros((LANES,), jnp.float32), jnp.zeros((LANES,), jnp.float32)))
        z_tot = z_acc[0]
        a_tot = a_acc[0]
        for lane in range(1, LANES):
            z_tot = z_tot + z_acc[lane]
            a_tot = a_tot + a_acc[lane]
        pv = (jnp.broadcast_to(a_tot, (LANES,))
              / jnp.broadcast_to(z_tot, (LANES,)))
        posts = jnp.where(lane_iota == r_local, pv, posts)
    p_v[...] = posts
    pltpu.sync_copy(p_v, post_hbm.at[wid])


_update_call = functools.partial(
    pl.kernel,
    out_type=[
        jax.ShapeDtypeStruct((MEM_SIZE, KEY_DIM), jnp.float32),
        jax.ShapeDtypeStruct((MEM_SIZE,), jnp.int32),
        jax.ShapeDtypeStruct((MEM_SIZE,), jnp.float32),
        jax.ShapeDtypeStruct((NW, LANES), jnp.float32),
    ],
    mesh=plsc.VectorSubcoreMesh(core_axis_name="c", subcore_axis_name="s"),
    scratch_types=[
        pltpu.VMEM((RPT,), jnp.int32),
        pltpu.VMEM((RPT,), jnp.float32),
        pltpu.VMEM((CHOOSE_K,), jnp.int32),
        pltpu.VMEM((CHOOSE_K,), jnp.int32),
        pltpu.VMEM((CHOOSE_K,), jnp.float32),
        pltpu.VMEM((CHOOSE_K,), jnp.int32),
        pltpu.VMEM((CHOOSE_K,), jnp.int32),
        pltpu.VMEM((LANES,), jnp.float32),
        pltpu.VMEM((2, KCH, KEY_DIM), jnp.float32),
        pltpu.SemaphoreType.DMA,
        pltpu.SemaphoreType.DMA,
        pltpu.SemaphoreType.DMA,
    ],
)(_update_kernel)


def kernel(x, label, W1, b1, W2, b2, mem_keys, mem_values, mem_ages):
    B = x.shape[0]
    xf = x.reshape(B, -1)

    q = pl.pallas_call(
        _q_kernel,
        out_shape=jax.ShapeDtypeStruct((B, KEY_DIM), jnp.float32),
    )(xf, W1, b1.reshape(1, FC_DIM), W2, b2.reshape(1, KEY_DIM))

    sim = pl.pallas_call(
        _sim_kernel,
        grid=(N_CHUNKS,),
        in_specs=[
            pl.BlockSpec((B, KEY_DIM), lambda i: (0, 0)),
            pl.BlockSpec((MEM_CHUNK, KEY_DIM), lambda i: (i, 0)),
        ],
        out_specs=pl.BlockSpec((B, MEM_CHUNK), lambda i: (0, i)),
        out_shape=jax.ShapeDtypeStruct((B, MEM_SIZE), jnp.float32),
    )(q, mem_keys)

    gmax = jnp.max(sim.reshape(B, N_GROUPS, GROUP), axis=-1)
    _, gidx = jax.lax.top_k(gmax, CHOOSE_K)
    cand_idx = (gidx[:, :, None] * GROUP
                + jnp.arange(GROUP, dtype=gidx.dtype)).reshape(B, CHOOSE_K * GROUP)
    cand = jnp.take_along_axis(sim, cand_idx, axis=1)
    topv, topj = jax.lax.top_k(cand, CHOOSE_K)
    topi = jnp.take_along_axis(cand_idx, topj, axis=1)

    nearest = topi[:, 0]
    match = jnp.take(mem_values, nearest, axis=0) == label
    merged = q + jnp.take(mem_keys, nearest, axis=0)
    merged = merged / (jnp.linalg.norm(merged, axis=1, keepdims=True) + 1e-8)
    _, oldest = jax.lax.top_k(mem_ages, B)
    write_idx = jnp.where(match, nearest, oldest).astype(jnp.int32)
    write_key = jnp.where(match[:, None], merged, q)

    jj = jnp.arange(CHOOSE_K, dtype=jnp.int32)
    eq = write_idx[:, None] == write_idx[None, :]
    winner = jnp.max(jnp.where(eq, jj[None, :], -1), axis=1)
    wk_eff = jnp.take(write_key, winner, axis=0)
    lab_eff = jnp.take(label.astype(jnp.int32), winner, axis=0)

    new_keys, new_values, new_ages, post_pad = _update_call(
        mem_keys, mem_values, mem_ages, wk_eff, write_idx, lab_eff,
        topv, topi)
    post_prob = post_pad[:, :BPT].reshape(B)
    return post_prob, new_keys, new_values, new_ages
